# baseline (speedup 1.0000x reference)
import functools

import jax
import jax.numpy as jnp
from jax import lax
from jax.experimental import pallas as pl
from jax.experimental.pallas import tpu as pltpu
from jax.experimental.pallas import tpu_sc as plsc

_INTERPRET = False
DP = jax.lax.Precision.DEFAULT

B = 32
P = 512
HP = jax.lax.Precision.HIGHEST
F32 = jnp.float32


def _dot(a, b, dims, prec):
    return jax.lax.dot_general(a, b, (dims, ((), ())),
                               precision=prec, preferred_element_type=F32)


def _mm(a, b, prec=HP):
    return _dot(a, b, ((1,), (0,)), prec)


def _mmT(a, b, prec=HP):
    return _dot(a, b, ((0,), (0,)), prec)


def _bn_from_stats(h, s, s2, n, g, be):
    mu = s / n
    var = s2 / n - mu * mu
    rstd = 1.0 / jnp.sqrt(var + 1e-5)
    return (h - mu[None, :]) * rstd[None, :] * g[None, :] + be[None, :]


def _gcn_block(adj, dinv, h, W, bvec):
    u = _mm(h, W, DP)
    t = _mmT(adj, dinv[:, None] * u)
    return dinv[:, None] * t + (dinv * dinv)[:, None] * u + bvec[None, :]


def _stat4(a, b):
    return jnp.concatenate([
        jnp.sum(a, axis=0).reshape(1, -1),
        jnp.sum(a * a, axis=0).reshape(1, -1),
        jnp.sum(b, axis=0).reshape(1, -1),
        jnp.sum(b * b, axis=0).reshape(1, -1)], axis=0)


def _t1_body(adj_ref, x_ref, pos_ref, W11_ref, b11_ref, Wp1_ref, bp1_ref,
             dinv_ref, x11_ref, s11_ref, stat_ref, adj2_ref):
    b = pl.program_id(0)
    adj = adj_ref[0]
    ones8 = jnp.ones((8, P), F32)
    deg = 1.0 + _mm(ones8, adj)[0]
    dinv = jnp.where(deg > 0, 1.0 / jnp.sqrt(jnp.where(deg > 0, deg, 1.0)), 0.0)
    dinv_ref[...] = dinv.reshape(1, 1, P)
    x11 = _gcn_block(adj, dinv, x_ref[...], W11_ref[...], b11_ref[...])
    s11 = _gcn_block(adj, dinv, pos_ref[...], Wp1_ref[...], bp1_ref[...])
    x11_ref[...] = x11
    s11_ref[...] = s11

    @pl.when(b == 0)
    def _():
        stat_ref[...] = jnp.zeros_like(stat_ref)
        adj2_ref[...] = jnp.zeros_like(adj2_ref)

    stat_ref[...] += _stat4(x11, s11)
    adj2_ref[...] += jnp.sum(adj * adj).reshape(1, 1)


def _mid_body(adj_ref, dinv_ref, xr_ref, sr_ref, stat_ref,
              Wx_ref, bx_ref, Ws_ref, bs_ref,
              gx_ref, bex_ref, gs_ref, bes_ref,
              sn_out_ref, xmax_ref, x_next_ref, s_next_ref,
              stat_out_ref, *, n_nodes):
    b = pl.program_id(0)
    adj = adj_ref[0]
    dinv = dinv_ref[0, 0]
    st = stat_ref[...]
    xn = _bn_from_stats(xr_ref[...], st[0], st[1], n_nodes, gx_ref[...], bex_ref[...])
    sn = _bn_from_stats(sr_ref[...], st[2], st[3], n_nodes, gs_ref[...], bes_ref[...])
    sn_out_ref[...] = sn
    xmax_ref[...] = jnp.max(xn, axis=0).reshape(1, 1, -1)
    x_next = _gcn_block(adj, dinv, xn, Wx_ref[...], bx_ref[...])
    s_next = _gcn_block(adj, dinv, sn, Ws_ref[...], bs_ref[...])
    x_next_ref[...] = x_next
    s_next_ref[...] = s_next

    @pl.when(b == 0)
    def _():
        stat_out_ref[...] = jnp.zeros_like(stat_out_ref)

    stat_out_ref[...] += _stat4(x_next, s_next)


def _d_body(adj_ref, c_ref, xr_ref, sr_ref, statx_ref, stats_ref,
            s11n_ref, s12n_ref,
            gx_ref, bex_ref, gs_ref, bes_ref, Wpf_ref, bpf_ref,
            xmax_ref, px_ref, padj_ref, scal_ref, *, n_nodes):
    b = pl.program_id(0)
    adj = adj_ref[0]
    cmat = c_ref[0]
    stx = statx_ref[...]
    sts = stats_ref[...]
    x13 = _bn_from_stats(xr_ref[...], stx[0], stx[1], n_nodes, gx_ref[...], bex_ref[...])
    s13 = _bn_from_stats(sr_ref[...], sts[0], sts[1], n_nodes, gs_ref[...], bes_ref[...])
    xmax_ref[...] = jnp.max(x13, axis=0).reshape(1, 1, -1)
    sc = jnp.concatenate([s11n_ref[...], s12n_ref[...], s13], axis=1)
    s1 = _mm(sc, Wpf_ref[...], DP) + bpf_ref[...][None, :]
    m = jnp.max(s1, axis=1, keepdims=True)
    a = s1 - m
    e = jnp.exp(a)
    z = jnp.sum(e, axis=1, keepdims=True)
    ssm = e / z
    t = _mm(adj, ssm, DP)
    u = _mm(cmat, ssm, DP)
    pa = _mmT(ssm, t, DP)
    padj_ref[0] = pa
    px_ref[0] = _mmT(ssm, x13, DP)
    k = ssm.shape[1]
    eye = (jax.lax.broadcasted_iota(jnp.int32, (k, k), 0) ==
           jax.lax.broadcasted_iota(jnp.int32, (k, k), 1)).astype(F32)
    cross = jnp.sum(pa * eye)
    mlsum = jnp.sum(_mmT(ssm, u, DP) * eye)
    g = _mmT(ssm, ssm)
    ssq = jnp.sum(g * g)
    entp = jnp.sum(jnp.log(z)) - jnp.sum(ssm * a)

    @pl.when(b == 0)
    def _():
        scal_ref[...] = jnp.zeros_like(scal_ref)

    scal_ref[...] += jnp.concatenate([
        cross.reshape(1, 1), mlsum.reshape(1, 1),
        ssq.reshape(1, 1), entp.reshape(1, 1)], axis=1)


def _e_body(px_ref, padj_ref, m11_ref, m12_ref, m13_ref, scal_ref, adj2_ref,
            W21_ref, b21_ref, g21_ref, be21_ref,
            W22_ref, b22_ref, g22_ref, be22_ref,
            W23_ref, b23_ref, g23_ref, be23_ref,
            Wf1_ref, bf1_ref, Wf2_ref, bf2_ref,
            out_ref, reg_ref, raw_ref, a1_ref, a2_ref, a3_ref, *, n_edges):
    k = 100
    eye = (jax.lax.broadcasted_iota(jnp.int32, (k, k), 0) ==
           jax.lax.broadcasted_iota(jnp.int32, (k, k), 1)).astype(F32)

    def dense_layer(h_in_ref, W, bvec, gvec, bevec, out_a_ref):
        def body(bb, carry):
            a2 = padj_ref[bb] + eye
            degc = jnp.sum(a2, axis=0)
            dinv = jnp.where(degc > 0,
                             1.0 / jnp.sqrt(jnp.where(degc > 0, degc, 1.0)), 0.0)
            hw = _mm(h_in_ref[bb], W, DP)
            t = _mmT(a2, dinv[:, None] * hw, DP)
            raw_ref[bb] = dinv[:, None] * t + bvec[None, :]
            return carry
        jax.lax.fori_loop(0, B, body, 0)
        raw = raw_ref[...].reshape(B * k, -1)
        mu = jnp.mean(raw, axis=0)
        var = jnp.mean((raw - mu[None, :]) ** 2, axis=0)
        a = (raw - mu[None, :]) / jnp.sqrt(var + 1e-5) * gvec[None, :] + bevec[None, :]
        out_a_ref[...] = a.reshape(B, k, -1)

    dense_layer(px_ref, W21_ref[...], b21_ref[...], g21_ref[...], be21_ref[...], a1_ref)
    dense_layer(a1_ref, W22_ref[...], b22_ref[...], g22_ref[...], be22_ref[...], a2_ref)
    dense_layer(a2_ref, W23_ref[...], b23_ref[...], g23_ref[...], be23_ref[...], a3_ref)

    x2 = jnp.concatenate([a1_ref[...], a2_ref[...], a3_ref[...]], axis=-1)
    x2max = jnp.max(x2, axis=1)
    conv = jnp.concatenate([m11_ref[...].reshape(B, -1), m12_ref[...].reshape(B, -1),
                            m13_ref[...].reshape(B, -1), x2max], axis=-1)
    h = _mm(conv, Wf1_ref[...], DP) + bf1_ref[...][None, :]
    out = _mm(jnp.maximum(h, 0.0), Wf2_ref[...], DP) + bf2_ref[...][None, :]
    out_ref[...] = out

    scal = scal_ref[...]
    cross = scal[0, 0]
    mlsum = scal[0, 1]
    ssq = scal[0, 2]
    entp = scal[0, 3]
    adj2 = adj2_ref[0, 0]
    link = jnp.sqrt(adj2 - 2.0 * cross + ssq) / (B * P * P)
    ent = entp / (B * P)
    ml = -mlsum / n_edges
    reg_ref[...] = (link + ent + ml).reshape(1, 1)


def _full(shape):
    nd = len(shape)
    return pl.BlockSpec(shape, lambda b: (0,) * nd)


def _pcall(body, grid, in_specs, out_specs, out_shape):
    return pl.pallas_call(
        body, grid=grid, in_specs=in_specs, out_specs=out_specs,
        out_shape=out_shape, interpret=_INTERPRET)


def kernel(x, pos, edge_index, edge_attr, num_graphs, params):
    p = params
    n = x.shape[0]
    n_edges = edge_index.shape[1]
    pg = n // B

    adj, cmat = _sc_build_adj(edge_index[0], edge_index[1], edge_attr)

    adj_spec = pl.BlockSpec((1, P, P), lambda b: (b, 0, 0))
    nodes = lambda d: pl.BlockSpec((P, d), lambda b: (b, 0))
    row = lambda d: pl.BlockSpec((1, 1, d), lambda b: (b, 0, 0))

    dinv, x11r, s11r, stat1, adj2 = _pcall(
        _t1_body, (B,),
        [adj_spec, nodes(3), nodes(44),
         _full((3, 30)), _full((30,)), _full((44, 30)), _full((30,))],
        [row(P), nodes(30), nodes(30), _full((4, 30)), _full((1, 1))],
        [jax.ShapeDtypeStruct((B, 1, P), F32),
         jax.ShapeDtypeStruct((n, 30), F32),
         jax.ShapeDtypeStruct((n, 30), F32),
         jax.ShapeDtypeStruct((4, 30), F32),
         jax.ShapeDtypeStruct((1, 1), F32)],
    )(adj, x, pos, p['W11'], p['b11'], p['Wp1'], p['bp1'])

    t2 = functools.partial(_mid_body, n_nodes=float(n))
    s11n, max11, x12r, s12r, stat2 = _pcall(
        t2, (B,),
        [adj_spec, row(P), nodes(30), nodes(30), _full((4, 30)),
         _full((30, 30)), _full((30,)), _full((30, 30)), _full((30,)),
         _full((30,)), _full((30,)), _full((30,)), _full((30,))],
        [nodes(30), row(30), nodes(30), nodes(30), _full((4, 30))],
        [jax.ShapeDtypeStruct((n, 30), F32),
         jax.ShapeDtypeStruct((B, 1, 30), F32),
         jax.ShapeDtypeStruct((n, 30), F32),
         jax.ShapeDtypeStruct((n, 30), F32),
         jax.ShapeDtypeStruct((4, 30), F32)],
    )(adj, dinv, x11r, s11r, stat1,
      p['W12'], p['b12'], p['Wp2'], p['bp2'],
      p['g_n11'], p['be_n11'], p['g_np1'], p['be_np1'])

    t3 = functools.partial(_mid3_caller, n=n)
    x13r, s13r, s12n, max12, statx3, stats3 = t3(
        adj, dinv, x12r, s12r, stat2,
        p['W13'], p['b13'], p['Wp3'], p['bp3'],
        p['g_n12'], p['be_n12'], p['g_np2'], p['be_np2'])

    d = functools.partial(_d_body, n_nodes=float(n))
    max13, px, padj, scal = _pcall(
        d, (B,),
        [adj_spec, adj_spec, nodes(30), nodes(100), _full((2, 30)), _full((2, 100)),
         nodes(30), nodes(30),
         _full((30,)), _full((30,)), _full((100,)), _full((100,)),
         _full((160, 100)), _full((100,))],
        [row(30), pl.BlockSpec((1, 100, 30), lambda b: (b, 0, 0)),
         pl.BlockSpec((1, 100, 100), lambda b: (b, 0, 0)), _full((1, 4))],
        [jax.ShapeDtypeStruct((B, 1, 30), F32),
         jax.ShapeDtypeStruct((B, 100, 30), F32),
         jax.ShapeDtypeStruct((B, 100, 100), F32),
         jax.ShapeDtypeStruct((1, 4), F32)],
    )(adj, cmat, x13r, s13r, statx3, stats3, s11n, s12n,
      p['g_n13'], p['be_n13'], p['g_np3'], p['be_np3'], p['Wpf'], p['bpf'])

    e = functools.partial(_e_body, n_edges=float(n_edges))
    out, reg = pl.pallas_call(
        e,
        out_shape=[jax.ShapeDtypeStruct((B, 6), F32),
                   jax.ShapeDtypeStruct((1, 1), F32)],
        scratch_shapes=[pltpu.VMEM((B, 100, 30), F32)] * 4,
        interpret=_INTERPRET,
    )(px, padj, max11, max12, max13, scal, adj2,
      p['W21'], p['b21'], p['g_n21'], p['be_n21'],
      p['W22'], p['b22'], p['g_n22'], p['be_n22'],
      p['W23'], p['b23'], p['g_n23'], p['be_n23'],
      p['Wf1'], p['bf1'], p['Wf2'], p['bf2'])

    return out, reg[0, 0]


def _t3_body(adj_ref, dinv_ref, xr_ref, sr_ref, stat_ref,
             Wx_ref, bx_ref, Ws_ref, bs_ref,
             gx_ref, bex_ref, gs_ref, bes_ref,
             x13_ref, s13_ref, s12n_ref, max12_ref, statx_ref, stats_ref,
             *, n_nodes):
    b = pl.program_id(0)
    adj = adj_ref[0]
    dinv = dinv_ref[0, 0]
    st = stat_ref[...]
    xn = _bn_from_stats(xr_ref[...], st[0], st[1], n_nodes, gx_ref[...], bex_ref[...])
    sn = _bn_from_stats(sr_ref[...], st[2], st[3], n_nodes, gs_ref[...], bes_ref[...])
    s12n_ref[...] = sn
    max12_ref[...] = jnp.max(xn, axis=0).reshape(1, 1, -1)
    x13 = _gcn_block(adj, dinv, xn, Wx_ref[...], bx_ref[...])
    s13 = _gcn_block(adj, dinv, sn, Ws_ref[...], bs_ref[...])
    x13_ref[...] = x13
    s13_ref[...] = s13

    @pl.when(b == 0)
    def _():
        statx_ref[...] = jnp.zeros_like(statx_ref)
        stats_ref[...] = jnp.zeros_like(stats_ref)

    statx_ref[...] += _stat4(x13, x13)[:2]
    stats_ref[...] += _stat4(s13, s13)[:2]


def _mid3_caller(adj, dinv, x12r, s12r, stat2, W13, b13, Wp3, bp3,
                 g12, be12, gp2, bep2, *, n):
    body = functools.partial(_t3_body, n_nodes=float(n))
    adj_spec = pl.BlockSpec((1, P, P), lambda b: (b, 0, 0))
    nodes = lambda d: pl.BlockSpec((P, d), lambda b: (b, 0))
    row = lambda d: pl.BlockSpec((1, 1, d), lambda b: (b, 0, 0))
    return _pcall(
        body, (B,),
        [adj_spec, row(P), nodes(30), nodes(30), _full((4, 30)),
         _full((30, 30)), _full((30,)), _full((30, 100)), _full((100,)),
         _full((30,)), _full((30,)), _full((30,)), _full((30,))],
        [nodes(30), nodes(100), nodes(30), row(30), _full((2, 30)), _full((2, 100))],
        [jax.ShapeDtypeStruct((n, 30), F32),
         jax.ShapeDtypeStruct((n, 100), F32),
         jax.ShapeDtypeStruct((n, 30), F32),
         jax.ShapeDtypeStruct((B, 1, 30), F32),
         jax.ShapeDtypeStruct((2, 30), F32),
         jax.ShapeDtypeStruct((2, 100), F32)],
    )(adj, dinv, x12r, s12r, stat2, W13, b13, Wp3, bp3, g12, be12, gp2, bep2)


_EPG = P * 16
_GPW = 2
_NW = 16 // _GPW
_EPT = _GPW * _EPG // 16
_WORDS = _GPW * P * P
_SHARE = _WORDS // 16
_NROW = _EPT // 128


def _sc_scatter_body(src_hbm, dst_hbm, ea_hbm, adj_hbm, c_hbm,
                     src_v, dst_v, ea_v, idx2, val2, ones2, zero_v,
                     adj_sh, c_sh):
    c_id = lax.axis_index("c")
    s_id = lax.axis_index("s")

    def zfill(i, carry):
        zero_v[pl.ds(i * 16, 16)] = jnp.zeros((16,), F32)
        return carry
    lax.fori_loop(0, _SHARE // 16, zfill, 0)
    for j in range(_NROW):
        ones2[j, :] = jnp.ones((128,), F32).reshape(128,)

    for w in range(_NW):
        pltpu.sync_copy(zero_v, adj_sh.at[pl.ds(s_id * _SHARE, _SHARE)])
        pltpu.sync_copy(zero_v, c_sh.at[pl.ds(s_id * _SHARE, _SHARE)])
        plsc.subcore_barrier()

        g_local = s_id // (16 // _GPW)
        part = s_id % (16 // _GPW)
        g = c_id * 16 + w * _GPW + g_local
        estart = g * _EPG + part * _EPT
        pltpu.sync_copy(src_hbm.at[pl.ds(estart, _EPT)], src_v)
        pltpu.sync_copy(dst_hbm.at[pl.ds(estart, _EPT)], dst_v)
        pltpu.sync_copy(ea_hbm.at[pl.ds(estart, _EPT)], ea_v)

        base = g_local * (P * P)
        for kk in range(_EPT // 16):
            sv = src_v[pl.ds(kk * 16, 16)]
            dv = dst_v[pl.ds(kk * 16, 16)]
            il = base + (sv & (P - 1)) * P + (dv & (P - 1))
            j, col = kk // 8, (kk % 8) * 16
            idx2[j, pl.ds(col, 16)] = il
            val2[j, pl.ds(col, 16)] = ea_v[pl.ds(kk * 16, 16)]

        for j in range(_NROW):
            pltpu.sync_copy(val2.at[j], adj_sh.at[idx2.at[j]], add=True)
        for j in range(_NROW):
            pltpu.sync_copy(ones2.at[j], c_sh.at[idx2.at[j]], add=True)
        plsc.subcore_barrier()

        out_base = (c_id * 16 + w * _GPW) * (P * P) + s_id * _SHARE
        pltpu.sync_copy(adj_sh.at[pl.ds(s_id * _SHARE, _SHARE)],
                        adj_hbm.at[pl.ds(out_base, _SHARE)])
        pltpu.sync_copy(c_sh.at[pl.ds(s_id * _SHARE, _SHARE)],
                        c_hbm.at[pl.ds(out_base, _SHARE)])
        plsc.subcore_barrier()


def _sc_build_adj(src, dst, ea):
    k = pl.kernel(
        _sc_scatter_body,
        out_type=[jax.ShapeDtypeStruct((B * P * P,), F32),
                  jax.ShapeDtypeStruct((B * P * P,), F32)],
        mesh=plsc.VectorSubcoreMesh(core_axis_name="c", subcore_axis_name="s"),
        scratch_types=[
            pltpu.VMEM((_EPT,), jnp.int32),
            pltpu.VMEM((_EPT,), jnp.int32),
            pltpu.VMEM((_EPT,), F32),
            pltpu.VMEM((_NROW, 128), jnp.int32),
            pltpu.VMEM((_NROW, 128), F32),
            pltpu.VMEM((_NROW, 128), F32),
            pltpu.VMEM((_SHARE,), F32),
            pltpu.VMEM_SHARED((_WORDS,), F32),
            pltpu.VMEM_SHARED((_WORDS,), F32),
        ],
    )
    adj_flat, c_flat = k(src, dst, ea)
    return adj_flat.reshape(B, P, P), c_flat.reshape(B, P, P)


# speedup vs baseline: 1.0450x; 1.0450x over previous
import functools

import jax
import jax.numpy as jnp
from jax import lax
from jax.experimental import pallas as pl
from jax.experimental.pallas import tpu as pltpu
from jax.experimental.pallas import tpu_sc as plsc

_INTERPRET = False
DP = jax.lax.Precision.DEFAULT
BF16 = jnp.bfloat16

B = 32
P = 512
HP = jax.lax.Precision.HIGHEST
F32 = jnp.float32


def _dot(a, b, dims, prec):
    return jax.lax.dot_general(a, b, (dims, ((), ())),
                               precision=prec, preferred_element_type=F32)


def _mm(a, b, prec=HP):
    return _dot(a, b, ((1,), (0,)), prec)


def _mmT(a, b, prec=HP):
    return _dot(a, b, ((0,), (0,)), prec)


def _bn_from_stats(h, s, s2, n, g, be):
    mu = s / n
    var = s2 / n - mu * mu
    rstd = 1.0 / jnp.sqrt(var + 1e-5)
    return (h - mu[None, :]) * rstd[None, :] * g[None, :] + be[None, :]


def _gcn_block(adj, dinv, h, W, bvec, msg_prec=HP):
    u = _mm(h, W, DP)
    t = _mmT(adj, dinv[:, None] * u, msg_prec)
    return dinv[:, None] * t + (dinv * dinv)[:, None] * u + bvec[None, :]


def _stat4(a, b):
    return jnp.concatenate([
        jnp.sum(a, axis=0).reshape(1, -1),
        jnp.sum(a * a, axis=0).reshape(1, -1),
        jnp.sum(b, axis=0).reshape(1, -1),
        jnp.sum(b * b, axis=0).reshape(1, -1)], axis=0)


def _t1_body(adj_ref, x_ref, pos_ref, W11_ref, b11_ref, Wp1_ref, bp1_ref,
             dinv_ref, x11_ref, s11_ref, stat_ref, adj2_ref, adjh_ref):
    b = pl.program_id(0)
    adj = adj_ref[0]
    ones8 = jnp.ones((8, P), F32)
    deg = 1.0 + _mm(ones8, adj)[0]
    dinv = jnp.where(deg > 0, 1.0 / jnp.sqrt(jnp.where(deg > 0, deg, 1.0)), 0.0)
    dinv_ref[...] = dinv.reshape(1, 1, P)
    x11 = _gcn_block(adj, dinv, x_ref[...], W11_ref[...], b11_ref[...])
    s11 = _gcn_block(adj, dinv, pos_ref[...], Wp1_ref[...], bp1_ref[...])
    x11_ref[...] = x11
    s11_ref[...] = s11

    @pl.when(b == 0)
    def _():
        stat_ref[...] = jnp.zeros_like(stat_ref)
        adj2_ref[...] = jnp.zeros_like(adj2_ref)

    stat_ref[...] += _stat4(x11, s11)
    adj2_ref[...] += jnp.sum(adj * adj).reshape(1, 1)
    adjh_ref[0] = adj.astype(BF16)


def _mid_body(adj_ref, dinv_ref, xr_ref, sr_ref, stat_ref,
              Wx_ref, bx_ref, Ws_ref, bs_ref,
              gx_ref, bex_ref, gs_ref, bes_ref,
              sn_out_ref, xmax_ref, x_next_ref, s_next_ref,
              stat_out_ref, *, n_nodes):
    b = pl.program_id(0)
    adj = adj_ref[0].astype(F32)
    dinv = dinv_ref[0, 0]
    st = stat_ref[...]
    xn = _bn_from_stats(xr_ref[...], st[0], st[1], n_nodes, gx_ref[...], bex_ref[...])
    sn = _bn_from_stats(sr_ref[...], st[2], st[3], n_nodes, gs_ref[...], bes_ref[...])
    sn_out_ref[...] = sn
    xmax_ref[...] = jnp.max(xn, axis=0).reshape(1, 1, -1)
    x_next = _gcn_block(adj, dinv, xn, Wx_ref[...], bx_ref[...])
    s_next = _gcn_block(adj, dinv, sn, Ws_ref[...], bs_ref[...])
    x_next_ref[...] = x_next
    s_next_ref[...] = s_next

    @pl.when(b == 0)
    def _():
        stat_out_ref[...] = jnp.zeros_like(stat_out_ref)

    stat_out_ref[...] += _stat4(x_next, s_next)


def _d_body(adj_ref, c_ref, xr_ref, sr_ref, statx_ref, stats_ref,
            s11n_ref, s12n_ref,
            gx_ref, bex_ref, gs_ref, bes_ref, Wpf_ref, bpf_ref,
            xmax_ref, px_ref, padj_ref, scal_ref, *, n_nodes):
    b = pl.program_id(0)
    adj = adj_ref[0].astype(F32)
    cmat = c_ref[0]
    stx = statx_ref[...]
    sts = stats_ref[...]
    x13 = _bn_from_stats(xr_ref[...], stx[0], stx[1], n_nodes, gx_ref[...], bex_ref[...])
    s13 = _bn_from_stats(sr_ref[...], sts[0], sts[1], n_nodes, gs_ref[...], bes_ref[...])
    xmax_ref[...] = jnp.max(x13, axis=0).reshape(1, 1, -1)
    sc = jnp.concatenate([s11n_ref[...], s12n_ref[...], s13], axis=1)
    s1 = _mm(sc, Wpf_ref[...], DP) + bpf_ref[...][None, :]
    m = jnp.max(s1, axis=1, keepdims=True)
    a = s1 - m
    e = jnp.exp(a)
    z = jnp.sum(e, axis=1, keepdims=True)
    ssm = e / z
    t = _mm(adj, ssm, DP)
    u = _mm(cmat, ssm, DP)
    pa = _mmT(ssm, t, DP)
    padj_ref[0] = pa
    px_ref[0] = _mmT(ssm, x13, DP)
    k = ssm.shape[1]
    eye = (jax.lax.broadcasted_iota(jnp.int32, (k, k), 0) ==
           jax.lax.broadcasted_iota(jnp.int32, (k, k), 1)).astype(F32)
    cross = jnp.sum(pa * eye)
    mlsum = jnp.sum(_mmT(ssm, u, DP) * eye)
    g = _mmT(ssm, ssm)
    ssq = jnp.sum(g * g)
    entp = jnp.sum(jnp.log(z)) - jnp.sum(ssm * a)

    @pl.when(b == 0)
    def _():
        scal_ref[...] = jnp.zeros_like(scal_ref)

    scal_ref[...] += jnp.concatenate([
        cross.reshape(1, 1), mlsum.reshape(1, 1),
        ssq.reshape(1, 1), entp.reshape(1, 1)], axis=1)


def _e_body(px_ref, padj_ref, m11_ref, m12_ref, m13_ref, scal_ref, adj2_ref,
            W21_ref, b21_ref, g21_ref, be21_ref,
            W22_ref, b22_ref, g22_ref, be22_ref,
            W23_ref, b23_ref, g23_ref, be23_ref,
            Wf1_ref, bf1_ref, Wf2_ref, bf2_ref,
            out_ref, reg_ref, raw_ref, a1_ref, a2_ref, a3_ref, *, n_edges):
    k = 100
    eye = (jax.lax.broadcasted_iota(jnp.int32, (k, k), 0) ==
           jax.lax.broadcasted_iota(jnp.int32, (k, k), 1)).astype(F32)

    def dense_layer(h_in_ref, W, bvec, gvec, bevec, out_a_ref):
        def body(bb, carry):
            a2 = padj_ref[bb] + eye
            degc = jnp.sum(a2, axis=0)
            dinv = jnp.where(degc > 0,
                             1.0 / jnp.sqrt(jnp.where(degc > 0, degc, 1.0)), 0.0)
            hw = _mm(h_in_ref[bb], W, DP)
            t = _mmT(a2, dinv[:, None] * hw, DP)
            raw_ref[bb] = dinv[:, None] * t + bvec[None, :]
            return carry
        jax.lax.fori_loop(0, B, body, 0)
        raw = raw_ref[...].reshape(B * k, -1)
        mu = jnp.mean(raw, axis=0)
        var = jnp.mean((raw - mu[None, :]) ** 2, axis=0)
        a = (raw - mu[None, :]) / jnp.sqrt(var + 1e-5) * gvec[None, :] + bevec[None, :]
        out_a_ref[...] = a.reshape(B, k, -1)

    dense_layer(px_ref, W21_ref[...], b21_ref[...], g21_ref[...], be21_ref[...], a1_ref)
    dense_layer(a1_ref, W22_ref[...], b22_ref[...], g22_ref[...], be22_ref[...], a2_ref)
    dense_layer(a2_ref, W23_ref[...], b23_ref[...], g23_ref[...], be23_ref[...], a3_ref)

    x2 = jnp.concatenate([a1_ref[...], a2_ref[...], a3_ref[...]], axis=-1)
    x2max = jnp.max(x2, axis=1)
    conv = jnp.concatenate([m11_ref[...].reshape(B, -1), m12_ref[...].reshape(B, -1),
                            m13_ref[...].reshape(B, -1), x2max], axis=-1)
    h = _mm(conv, Wf1_ref[...], DP) + bf1_ref[...][None, :]
    out = _mm(jnp.maximum(h, 0.0), Wf2_ref[...], DP) + bf2_ref[...][None, :]
    out_ref[...] = out

    scal = scal_ref[...]
    cross = scal[0, 0]
    mlsum = scal[0, 1]
    ssq = scal[0, 2]
    entp = scal[0, 3]
    adj2 = adj2_ref[0, 0]
    link = jnp.sqrt(adj2 - 2.0 * cross + ssq) / (B * P * P)
    ent = entp / (B * P)
    ml = -mlsum / n_edges
    reg_ref[...] = (link + ent + ml).reshape(1, 1)


def _full(shape):
    nd = len(shape)
    return pl.BlockSpec(shape, lambda b: (0,) * nd)


def _pcall(body, grid, in_specs, out_specs, out_shape):
    return pl.pallas_call(
        body, grid=grid, in_specs=in_specs, out_specs=out_specs,
        out_shape=out_shape, interpret=_INTERPRET)


def kernel(x, pos, edge_index, edge_attr, num_graphs, params):
    p = params
    n = x.shape[0]
    n_edges = edge_index.shape[1]
    pg = n // B

    adj, cmat = _sc_build_adj(edge_index[0], edge_index[1], edge_attr)

    adj_spec = pl.BlockSpec((1, P, P), lambda b: (b, 0, 0))
    nodes = lambda d: pl.BlockSpec((P, d), lambda b: (b, 0))
    row = lambda d: pl.BlockSpec((1, 1, d), lambda b: (b, 0, 0))

    dinv, x11r, s11r, stat1, adj2, adjh = _pcall(
        _t1_body, (B,),
        [adj_spec, nodes(3), nodes(44),
         _full((3, 30)), _full((30,)), _full((44, 30)), _full((30,))],
        [row(P), nodes(30), nodes(30), _full((4, 30)), _full((1, 1)), adj_spec],
        [jax.ShapeDtypeStruct((B, 1, P), F32),
         jax.ShapeDtypeStruct((n, 30), F32),
         jax.ShapeDtypeStruct((n, 30), F32),
         jax.ShapeDtypeStruct((4, 30), F32),
         jax.ShapeDtypeStruct((1, 1), F32),
         jax.ShapeDtypeStruct((B, P, P), BF16)],
    )(adj, x, pos, p['W11'], p['b11'], p['Wp1'], p['bp1'])

    t2 = functools.partial(_mid_body, n_nodes=float(n))
    s11n, max11, x12r, s12r, stat2 = _pcall(
        t2, (B,),
        [adj_spec, row(P), nodes(30), nodes(30), _full((4, 30)),
         _full((30, 30)), _full((30,)), _full((30, 30)), _full((30,)),
         _full((30,)), _full((30,)), _full((30,)), _full((30,))],
        [nodes(30), row(30), nodes(30), nodes(30), _full((4, 30))],
        [jax.ShapeDtypeStruct((n, 30), F32),
         jax.ShapeDtypeStruct((B, 1, 30), F32),
         jax.ShapeDtypeStruct((n, 30), F32),
         jax.ShapeDtypeStruct((n, 30), F32),
         jax.ShapeDtypeStruct((4, 30), F32)],
    )(adjh, dinv, x11r, s11r, stat1,
      p['W12'], p['b12'], p['Wp2'], p['bp2'],
      p['g_n11'], p['be_n11'], p['g_np1'], p['be_np1'])

    t3 = functools.partial(_mid3_caller, n=n)
    x13r, s13r, s12n, max12, statx3, stats3 = t3(
        adjh, dinv, x12r, s12r, stat2,
        p['W13'], p['b13'], p['Wp3'], p['bp3'],
        p['g_n12'], p['be_n12'], p['g_np2'], p['be_np2'])

    d = functools.partial(_d_body, n_nodes=float(n))
    max13, px, padj, scal = _pcall(
        d, (B,),
        [adj_spec, adj_spec, nodes(30), nodes(100), _full((2, 30)), _full((2, 100)),
         nodes(30), nodes(30),
         _full((30,)), _full((30,)), _full((100,)), _full((100,)),
         _full((160, 100)), _full((100,))],
        [row(30), pl.BlockSpec((1, 100, 30), lambda b: (b, 0, 0)),
         pl.BlockSpec((1, 100, 100), lambda b: (b, 0, 0)), _full((1, 4))],
        [jax.ShapeDtypeStruct((B, 1, 30), F32),
         jax.ShapeDtypeStruct((B, 100, 30), F32),
         jax.ShapeDtypeStruct((B, 100, 100), F32),
         jax.ShapeDtypeStruct((1, 4), F32)],
    )(adjh, cmat, x13r, s13r, statx3, stats3, s11n, s12n,
      p['g_n13'], p['be_n13'], p['g_np3'], p['be_np3'], p['Wpf'], p['bpf'])

    e = functools.partial(_e_body, n_edges=float(n_edges))
    out, reg = pl.pallas_call(
        e,
        out_shape=[jax.ShapeDtypeStruct((B, 6), F32),
                   jax.ShapeDtypeStruct((1, 1), F32)],
        scratch_shapes=[pltpu.VMEM((B, 100, 30), F32)] * 4,
        interpret=_INTERPRET,
    )(px, padj, max11, max12, max13, scal, adj2,
      p['W21'], p['b21'], p['g_n21'], p['be_n21'],
      p['W22'], p['b22'], p['g_n22'], p['be_n22'],
      p['W23'], p['b23'], p['g_n23'], p['be_n23'],
      p['Wf1'], p['bf1'], p['Wf2'], p['bf2'])

    return out, reg[0, 0]


def _t3_body(adj_ref, dinv_ref, xr_ref, sr_ref, stat_ref,
             Wx_ref, bx_ref, Ws_ref, bs_ref,
             gx_ref, bex_ref, gs_ref, bes_ref,
             x13_ref, s13_ref, s12n_ref, max12_ref, statx_ref, stats_ref,
             *, n_nodes):
    b = pl.program_id(0)
    adj = adj_ref[0].astype(F32)
    dinv = dinv_ref[0, 0]
    st = stat_ref[...]
    xn = _bn_from_stats(xr_ref[...], st[0], st[1], n_nodes, gx_ref[...], bex_ref[...])
    sn = _bn_from_stats(sr_ref[...], st[2], st[3], n_nodes, gs_ref[...], bes_ref[...])
    s12n_ref[...] = sn
    max12_ref[...] = jnp.max(xn, axis=0).reshape(1, 1, -1)
    x13 = _gcn_block(adj, dinv, xn, Wx_ref[...], bx_ref[...])
    s13 = _gcn_block(adj, dinv, sn, Ws_ref[...], bs_ref[...])
    x13_ref[...] = x13
    s13_ref[...] = s13

    @pl.when(b == 0)
    def _():
        statx_ref[...] = jnp.zeros_like(statx_ref)
        stats_ref[...] = jnp.zeros_like(stats_ref)

    statx_ref[...] += _stat4(x13, x13)[:2]
    stats_ref[...] += _stat4(s13, s13)[:2]


def _mid3_caller(adj, dinv, x12r, s12r, stat2, W13, b13, Wp3, bp3,
                 g12, be12, gp2, bep2, *, n):
    body = functools.partial(_t3_body, n_nodes=float(n))
    adj_spec = pl.BlockSpec((1, P, P), lambda b: (b, 0, 0))
    nodes = lambda d: pl.BlockSpec((P, d), lambda b: (b, 0))
    row = lambda d: pl.BlockSpec((1, 1, d), lambda b: (b, 0, 0))
    return _pcall(
        body, (B,),
        [adj_spec, row(P), nodes(30), nodes(30), _full((4, 30)),
         _full((30, 30)), _full((30,)), _full((30, 100)), _full((100,)),
         _full((30,)), _full((30,)), _full((30,)), _full((30,))],
        [nodes(30), nodes(100), nodes(30), row(30), _full((2, 30)), _full((2, 100))],
        [jax.ShapeDtypeStruct((n, 30), F32),
         jax.ShapeDtypeStruct((n, 100), F32),
         jax.ShapeDtypeStruct((n, 30), F32),
         jax.ShapeDtypeStruct((B, 1, 30), F32),
         jax.ShapeDtypeStruct((2, 30), F32),
         jax.ShapeDtypeStruct((2, 100), F32)],
    )(adj, dinv, x12r, s12r, stat2, W13, b13, Wp3, bp3, g12, be12, gp2, bep2)


_EPG = P * 16
_GPW = 2
_NW = 16 // _GPW
_EPT = _GPW * _EPG // 16
_WORDS = _GPW * P * P
_SHARE = _WORDS // 16
_NROW = _EPT // 128


def _sc_scatter_body(src_hbm, dst_hbm, ea_hbm, adj_hbm, c_hbm,
                     src_v, dst_v, ea_v, idx2, val2, ones2, zero_v,
                     adj_sh, c_sh, sem_z, sem_st, sem_sc, sem_d):
    c_id = lax.axis_index("c")
    s_id = lax.axis_index("s")

    def zfill(i, carry):
        zero_v[pl.ds(i * 16, 16)] = jnp.zeros((16,), F32)
        return carry
    lax.fori_loop(0, _SHARE // 16, zfill, 0)
    for j in range(_NROW):
        ones2[j, :] = jnp.ones((128,), F32).reshape(128,)

    for w in range(_NW):
        z1 = pltpu.async_copy(zero_v, adj_sh.at[pl.ds(s_id * _SHARE, _SHARE)], sem_z)
        z2 = pltpu.async_copy(zero_v, c_sh.at[pl.ds(s_id * _SHARE, _SHARE)], sem_z)

        g_local = s_id // (16 // _GPW)
        part = s_id % (16 // _GPW)
        g = c_id * 16 + w * _GPW + g_local
        estart = g * _EPG + part * _EPT
        st1 = pltpu.async_copy(src_hbm.at[pl.ds(estart, _EPT)], src_v, sem_st)
        st2 = pltpu.async_copy(dst_hbm.at[pl.ds(estart, _EPT)], dst_v, sem_st)
        st3 = pltpu.async_copy(ea_hbm.at[pl.ds(estart, _EPT)], ea_v, sem_st)
        st1.wait(); st2.wait(); st3.wait()

        base = g_local * (P * P)
        for kk in range(_EPT // 16):
            sv = src_v[pl.ds(kk * 16, 16)]
            dv = dst_v[pl.ds(kk * 16, 16)]
            il = base + (sv & (P - 1)) * P + (dv & (P - 1))
            j, col = kk // 8, (kk % 8) * 16
            idx2[j, pl.ds(col, 16)] = il
            val2[j, pl.ds(col, 16)] = ea_v[pl.ds(kk * 16, 16)]

        z1.wait(); z2.wait()
        plsc.subcore_barrier()

        descs = []
        for j in range(_NROW):
            descs.append(pltpu.async_copy(
                val2.at[j], adj_sh.at[idx2.at[j]], sem_sc, add=True))
            descs.append(pltpu.async_copy(
                ones2.at[j], c_sh.at[idx2.at[j]], sem_sc, add=True))
        for d in descs:
            d.wait()
        plsc.subcore_barrier()

        out_base = (c_id * 16 + w * _GPW) * (P * P) + s_id * _SHARE
        d1 = pltpu.async_copy(adj_sh.at[pl.ds(s_id * _SHARE, _SHARE)],
                              adj_hbm.at[pl.ds(out_base, _SHARE)], sem_d)
        d2 = pltpu.async_copy(c_sh.at[pl.ds(s_id * _SHARE, _SHARE)],
                              c_hbm.at[pl.ds(out_base, _SHARE)], sem_d)
        d1.wait(); d2.wait()
        plsc.subcore_barrier()


def _sc_build_adj(src, dst, ea):
    k = pl.kernel(
        _sc_scatter_body,
        out_type=[jax.ShapeDtypeStruct((B * P * P,), F32),
                  jax.ShapeDtypeStruct((B * P * P,), F32)],
        mesh=plsc.VectorSubcoreMesh(core_axis_name="c", subcore_axis_name="s"),
        scratch_types=[
            pltpu.VMEM((_EPT,), jnp.int32),
            pltpu.VMEM((_EPT,), jnp.int32),
            pltpu.VMEM((_EPT,), F32),
            pltpu.VMEM((_NROW, 128), jnp.int32),
            pltpu.VMEM((_NROW, 128), F32),
            pltpu.VMEM((_NROW, 128), F32),
            pltpu.VMEM((_SHARE,), F32),
            pltpu.VMEM_SHARED((_WORDS,), F32),
            pltpu.VMEM_SHARED((_WORDS,), F32),
            pltpu.SemaphoreType.DMA,
            pltpu.SemaphoreType.DMA,
            pltpu.SemaphoreType.DMA,
            pltpu.SemaphoreType.DMA,
        ],
    )
    adj_flat, c_flat = k(src, dst, ea)
    return adj_flat.reshape(B, P, P), c_flat.reshape(B, P, P)


# speedup vs baseline: 1.4146x; 1.3536x over previous
import functools

import jax
import jax.numpy as jnp
from jax import lax
from jax.experimental import pallas as pl
from jax.experimental.pallas import tpu as pltpu
from jax.experimental.pallas import tpu_sc as plsc

_INTERPRET = False
DP = jax.lax.Precision.DEFAULT
BF16 = jnp.bfloat16

B = 32
P = 512
HP = jax.lax.Precision.HIGHEST
F32 = jnp.float32


def _dot(a, b, dims, prec):
    return jax.lax.dot_general(a, b, (dims, ((), ())),
                               precision=prec, preferred_element_type=F32)


def _mm(a, b, prec=HP):
    return _dot(a, b, ((1,), (0,)), prec)


def _mmT(a, b, prec=HP):
    return _dot(a, b, ((0,), (0,)), prec)


def _bn_from_stats(h, s, s2, n, g, be):
    mu = s / n
    var = s2 / n - mu * mu
    rstd = 1.0 / jnp.sqrt(var + 1e-5)
    return (h - mu[None, :]) * rstd[None, :] * g[None, :] + be[None, :]


def _gcn_block(adj, dinv, h, W, bvec, msg_prec=DP):
    u = _mm(h, W, DP)
    t = _mmT(adj, dinv[:, None] * u, msg_prec)
    return dinv[:, None] * t + (dinv * dinv)[:, None] * u + bvec[None, :]


def _gcn_pair(adj, dinv, hx, Wx, bx, hs, Ws, bs):
    dx = Wx.shape[1]
    u = jnp.concatenate([_mm(hx, Wx, DP), _mm(hs, Ws, DP)], axis=1)
    t = _mmT(adj, dinv[:, None] * u, DP)
    out = dinv[:, None] * t + (dinv * dinv)[:, None] * u
    return out[:, :dx] + bx[None, :], out[:, dx:] + bs[None, :]


def _stat4(a, b):
    return jnp.concatenate([
        jnp.sum(a, axis=0).reshape(1, -1),
        jnp.sum(a * a, axis=0).reshape(1, -1),
        jnp.sum(b, axis=0).reshape(1, -1),
        jnp.sum(b * b, axis=0).reshape(1, -1)], axis=0)


def _t1_body(adj_ref, x_ref, pos_ref, W11_ref, b11_ref, Wp1_ref, bp1_ref,
             dinv_ref, x11_ref, s11_ref, stat_ref, adj2_ref, adjh_ref):
    b = pl.program_id(0)
    adj = adj_ref[0]
    ones8 = jnp.ones((8, P), F32)
    deg = 1.0 + _mm(ones8, adj)[0]
    dinv = jnp.where(deg > 0, 1.0 / jnp.sqrt(jnp.where(deg > 0, deg, 1.0)), 0.0)
    dinv_ref[...] = dinv.reshape(1, 1, P)
    x11, s11 = _gcn_pair(adj, dinv, x_ref[...], W11_ref[...], b11_ref[...],
                         pos_ref[...], Wp1_ref[...], bp1_ref[...])
    x11_ref[...] = x11
    s11_ref[...] = s11

    @pl.when(b == 0)
    def _():
        stat_ref[...] = jnp.zeros_like(stat_ref)
        adj2_ref[...] = jnp.zeros_like(adj2_ref)

    stat_ref[...] += _stat4(x11, s11)
    adj2_ref[...] += jnp.sum(adj * adj).reshape(1, 1)
    adjh_ref[0] = adj.astype(BF16)


def _mid_body(adj_ref, dinv_ref, xr_ref, sr_ref, stat_ref,
              Wx_ref, bx_ref, Ws_ref, bs_ref,
              gx_ref, bex_ref, gs_ref, bes_ref,
              sn_out_ref, xmax_ref, x_next_ref, s_next_ref,
              stat_out_ref, *, n_nodes):
    b = pl.program_id(0)
    adj = adj_ref[0].astype(F32)
    dinv = dinv_ref[0, 0]
    st = stat_ref[...]
    xn = _bn_from_stats(xr_ref[...], st[0], st[1], n_nodes, gx_ref[...], bex_ref[...])
    sn = _bn_from_stats(sr_ref[...], st[2], st[3], n_nodes, gs_ref[...], bes_ref[...])
    sn_out_ref[...] = sn
    xmax_ref[...] = jnp.max(xn, axis=0).reshape(1, 1, -1)
    x_next, s_next = _gcn_pair(adj, dinv, xn, Wx_ref[...], bx_ref[...],
                               sn, Ws_ref[...], bs_ref[...])
    x_next_ref[...] = x_next
    s_next_ref[...] = s_next

    @pl.when(b == 0)
    def _():
        stat_out_ref[...] = jnp.zeros_like(stat_out_ref)

    stat_out_ref[...] += _stat4(x_next, s_next)


def _d_body(adj_ref, c_ref, xr_ref, sr_ref, statx_ref, stats_ref,
            s11n_ref, s12n_ref,
            gx_ref, bex_ref, gs_ref, bes_ref, Wpf_ref, bpf_ref,
            xmax_ref, px_ref, padj_ref, scal_ref, *, n_nodes):
    b = pl.program_id(0)
    adj = adj_ref[0].astype(F32)
    cmat = c_ref[0]
    stx = statx_ref[...]
    sts = stats_ref[...]
    x13 = _bn_from_stats(xr_ref[...], stx[0], stx[1], n_nodes, gx_ref[...], bex_ref[...])
    s13 = _bn_from_stats(sr_ref[...], sts[0], sts[1], n_nodes, gs_ref[...], bes_ref[...])
    xmax_ref[...] = jnp.max(x13, axis=0).reshape(1, 1, -1)
    sc = jnp.concatenate([s11n_ref[...], s12n_ref[...], s13], axis=1)
    s1 = _mm(sc, Wpf_ref[...], DP) + bpf_ref[...][None, :]
    m = jnp.max(s1, axis=1, keepdims=True)
    a = s1 - m
    e = jnp.exp(a)
    z = jnp.sum(e, axis=1, keepdims=True)
    ssm = e / z
    t = _mm(adj, ssm, DP)
    u = _mm(cmat, ssm, DP)
    pa = _mmT(ssm, t, DP)
    padj_ref[0] = pa
    px_ref[0] = _mmT(ssm, x13, DP)
    k = ssm.shape[1]
    eye = (jax.lax.broadcasted_iota(jnp.int32, (k, k), 0) ==
           jax.lax.broadcasted_iota(jnp.int32, (k, k), 1)).astype(F32)
    cross = jnp.sum(pa * eye)
    mlsum = jnp.sum(_mmT(ssm, u, DP) * eye)
    g = _mmT(ssm, ssm)
    ssq = jnp.sum(g * g)
    entp = jnp.sum(jnp.log(z)) - jnp.sum(ssm * a)

    @pl.when(b == 0)
    def _():
        scal_ref[...] = jnp.zeros_like(scal_ref)

    scal_ref[...] += jnp.concatenate([
        cross.reshape(1, 1), mlsum.reshape(1, 1),
        ssq.reshape(1, 1), entp.reshape(1, 1)], axis=1)


def _e_body(px_ref, padj_ref, m11_ref, m12_ref, m13_ref, scal_ref, adj2_ref,
            W21_ref, b21_ref, g21_ref, be21_ref,
            W22_ref, b22_ref, g22_ref, be22_ref,
            W23_ref, b23_ref, g23_ref, be23_ref,
            Wf1_ref, bf1_ref, Wf2_ref, bf2_ref,
            out_ref, reg_ref, raw_ref, a1_ref, a2_ref, a3_ref, *, n_edges):
    k = 100
    eye = (jax.lax.broadcasted_iota(jnp.int32, (k, k), 0) ==
           jax.lax.broadcasted_iota(jnp.int32, (k, k), 1)).astype(F32)

    def dense_layer(h_in_ref, W, bvec, gvec, bevec, out_a_ref):
        def body(bb, carry):
            a2 = padj_ref[bb] + eye
            degc = jnp.sum(a2, axis=0)
            dinv = jnp.where(degc > 0,
                             1.0 / jnp.sqrt(jnp.where(degc > 0, degc, 1.0)), 0.0)
            hw = _mm(h_in_ref[bb], W, DP)
            t = _mmT(a2, dinv[:, None] * hw, DP)
            raw_ref[bb] = dinv[:, None] * t + bvec[None, :]
            return carry
        jax.lax.fori_loop(0, B, body, 0)
        raw = raw_ref[...].reshape(B * k, -1)
        mu = jnp.mean(raw, axis=0)
        var = jnp.mean((raw - mu[None, :]) ** 2, axis=0)
        a = (raw - mu[None, :]) / jnp.sqrt(var + 1e-5) * gvec[None, :] + bevec[None, :]
        out_a_ref[...] = a.reshape(B, k, -1)

    dense_layer(px_ref, W21_ref[...], b21_ref[...], g21_ref[...], be21_ref[...], a1_ref)
    dense_layer(a1_ref, W22_ref[...], b22_ref[...], g22_ref[...], be22_ref[...], a2_ref)
    dense_layer(a2_ref, W23_ref[...], b23_ref[...], g23_ref[...], be23_ref[...], a3_ref)

    x2 = jnp.concatenate([a1_ref[...], a2_ref[...], a3_ref[...]], axis=-1)
    x2max = jnp.max(x2, axis=1)
    conv = jnp.concatenate([m11_ref[...].reshape(B, -1), m12_ref[...].reshape(B, -1),
                            m13_ref[...].reshape(B, -1), x2max], axis=-1)
    h = _mm(conv, Wf1_ref[...], DP) + bf1_ref[...][None, :]
    out = _mm(jnp.maximum(h, 0.0), Wf2_ref[...], DP) + bf2_ref[...][None, :]
    out_ref[...] = out

    scal = scal_ref[...]
    cross = scal[0, 0]
    mlsum = scal[0, 1]
    ssq = scal[0, 2]
    entp = scal[0, 3]
    adj2 = adj2_ref[0, 0]
    link = jnp.sqrt(adj2 - 2.0 * cross + ssq) / (B * P * P)
    ent = entp / (B * P)
    ml = -mlsum / n_edges
    reg_ref[...] = (link + ent + ml).reshape(1, 1)


def _full(shape):
    nd = len(shape)
    return pl.BlockSpec(shape, lambda b: (0,) * nd)


def _pcall(body, grid, in_specs, out_specs, out_shape):
    return pl.pallas_call(
        body, grid=grid, in_specs=in_specs, out_specs=out_specs,
        out_shape=out_shape, interpret=_INTERPRET)


def kernel(x, pos, edge_index, edge_attr, num_graphs, params):
    p = params
    n = x.shape[0]
    n_edges = edge_index.shape[1]
    pg = n // B

    adj, cmat = _sc_build_adj(edge_index[0], edge_index[1], edge_attr)

    adj_spec = pl.BlockSpec((1, P, P), lambda b: (b, 0, 0))
    nodes = lambda d: pl.BlockSpec((P, d), lambda b: (b, 0))
    row = lambda d: pl.BlockSpec((1, 1, d), lambda b: (b, 0, 0))

    dinv, x11r, s11r, stat1, adj2, adjh = _pcall(
        _t1_body, (B,),
        [adj_spec, nodes(3), nodes(44),
         _full((3, 30)), _full((30,)), _full((44, 30)), _full((30,))],
        [row(P), nodes(30), nodes(30), _full((4, 30)), _full((1, 1)), adj_spec],
        [jax.ShapeDtypeStruct((B, 1, P), F32),
         jax.ShapeDtypeStruct((n, 30), F32),
         jax.ShapeDtypeStruct((n, 30), F32),
         jax.ShapeDtypeStruct((4, 30), F32),
         jax.ShapeDtypeStruct((1, 1), F32),
         jax.ShapeDtypeStruct((B, P, P), BF16)],
    )(adj, x, pos, p['W11'], p['b11'], p['Wp1'], p['bp1'])

    t2 = functools.partial(_mid_body, n_nodes=float(n))
    s11n, max11, x12r, s12r, stat2 = _pcall(
        t2, (B,),
        [adj_spec, row(P), nodes(30), nodes(30), _full((4, 30)),
         _full((30, 30)), _full((30,)), _full((30, 30)), _full((30,)),
         _full((30,)), _full((30,)), _full((30,)), _full((30,))],
        [nodes(30), row(30), nodes(30), nodes(30), _full((4, 30))],
        [jax.ShapeDtypeStruct((n, 30), F32),
         jax.ShapeDtypeStruct((B, 1, 30), F32),
         jax.ShapeDtypeStruct((n, 30), F32),
         jax.ShapeDtypeStruct((n, 30), F32),
         jax.ShapeDtypeStruct((4, 30), F32)],
    )(adjh, dinv, x11r, s11r, stat1,
      p['W12'], p['b12'], p['Wp2'], p['bp2'],
      p['g_n11'], p['be_n11'], p['g_np1'], p['be_np1'])

    t3 = functools.partial(_mid3_caller, n=n)
    x13r, s13r, s12n, max12, statx3, stats3 = t3(
        adjh, dinv, x12r, s12r, stat2,
        p['W13'], p['b13'], p['Wp3'], p['bp3'],
        p['g_n12'], p['be_n12'], p['g_np2'], p['be_np2'])

    d = functools.partial(_d_body, n_nodes=float(n))
    max13, px, padj, scal = _pcall(
        d, (B,),
        [adj_spec, adj_spec, nodes(30), nodes(100), _full((2, 30)), _full((2, 100)),
         nodes(30), nodes(30),
         _full((30,)), _full((30,)), _full((100,)), _full((100,)),
         _full((160, 100)), _full((100,))],
        [row(30), pl.BlockSpec((1, 100, 30), lambda b: (b, 0, 0)),
         pl.BlockSpec((1, 100, 100), lambda b: (b, 0, 0)), _full((1, 4))],
        [jax.ShapeDtypeStruct((B, 1, 30), F32),
         jax.ShapeDtypeStruct((B, 100, 30), F32),
         jax.ShapeDtypeStruct((B, 100, 100), F32),
         jax.ShapeDtypeStruct((1, 4), F32)],
    )(adjh, cmat, x13r, s13r, statx3, stats3, s11n, s12n,
      p['g_n13'], p['be_n13'], p['g_np3'], p['be_np3'], p['Wpf'], p['bpf'])

    e = functools.partial(_e_body, n_edges=float(n_edges))
    out, reg = pl.pallas_call(
        e,
        out_shape=[jax.ShapeDtypeStruct((B, 6), F32),
                   jax.ShapeDtypeStruct((1, 1), F32)],
        scratch_shapes=[pltpu.VMEM((B, 100, 30), F32)] * 4,
        interpret=_INTERPRET,
    )(px, padj, max11, max12, max13, scal, adj2,
      p['W21'], p['b21'], p['g_n21'], p['be_n21'],
      p['W22'], p['b22'], p['g_n22'], p['be_n22'],
      p['W23'], p['b23'], p['g_n23'], p['be_n23'],
      p['Wf1'], p['bf1'], p['Wf2'], p['bf2'])

    return out, reg[0, 0]


def _t3_body(adj_ref, dinv_ref, xr_ref, sr_ref, stat_ref,
             Wx_ref, bx_ref, Ws_ref, bs_ref,
             gx_ref, bex_ref, gs_ref, bes_ref,
             x13_ref, s13_ref, s12n_ref, max12_ref, statx_ref, stats_ref,
             *, n_nodes):
    b = pl.program_id(0)
    adj = adj_ref[0].astype(F32)
    dinv = dinv_ref[0, 0]
    st = stat_ref[...]
    xn = _bn_from_stats(xr_ref[...], st[0], st[1], n_nodes, gx_ref[...], bex_ref[...])
    sn = _bn_from_stats(sr_ref[...], st[2], st[3], n_nodes, gs_ref[...], bes_ref[...])
    s12n_ref[...] = sn
    max12_ref[...] = jnp.max(xn, axis=0).reshape(1, 1, -1)
    x13 = _gcn_block(adj, dinv, xn, Wx_ref[...], bx_ref[...])
    s13 = _gcn_block(adj, dinv, sn, Ws_ref[...], bs_ref[...])
    x13_ref[...] = x13
    s13_ref[...] = s13

    @pl.when(b == 0)
    def _():
        statx_ref[...] = jnp.zeros_like(statx_ref)
        stats_ref[...] = jnp.zeros_like(stats_ref)

    statx_ref[...] += _stat4(x13, x13)[:2]
    stats_ref[...] += _stat4(s13, s13)[:2]


def _mid3_caller(adj, dinv, x12r, s12r, stat2, W13, b13, Wp3, bp3,
                 g12, be12, gp2, bep2, *, n):
    body = functools.partial(_t3_body, n_nodes=float(n))
    adj_spec = pl.BlockSpec((1, P, P), lambda b: (b, 0, 0))
    nodes = lambda d: pl.BlockSpec((P, d), lambda b: (b, 0))
    row = lambda d: pl.BlockSpec((1, 1, d), lambda b: (b, 0, 0))
    return _pcall(
        body, (B,),
        [adj_spec, row(P), nodes(30), nodes(30), _full((4, 30)),
         _full((30, 30)), _full((30,)), _full((30, 100)), _full((100,)),
         _full((30,)), _full((30,)), _full((30,)), _full((30,))],
        [nodes(30), nodes(100), nodes(30), row(30), _full((2, 30)), _full((2, 100))],
        [jax.ShapeDtypeStruct((n, 30), F32),
         jax.ShapeDtypeStruct((n, 100), F32),
         jax.ShapeDtypeStruct((n, 30), F32),
         jax.ShapeDtypeStruct((B, 1, 30), F32),
         jax.ShapeDtypeStruct((2, 30), F32),
         jax.ShapeDtypeStruct((2, 100), F32)],
    )(adj, dinv, x12r, s12r, stat2, W13, b13, Wp3, bp3, g12, be12, gp2, bep2)


_EPG = P * 16
_GPW = 2
_NW = 16 // _GPW
_EPT = _GPW * _EPG // 16
_WORDS = _GPW * P * P
_SHARE = _WORDS // 16
_NROW = _EPT // 128


def _sc_scatter_body(src_hbm, dst_hbm, ea_hbm, adj_hbm, c_hbm,
                     src_v, dst_v, ea_v, idx2, val2, ones2, zero_v,
                     adj_sh, c_sh, sem_z, sem_st, sem_sc, sem_d):
    c_id = lax.axis_index("c")
    s_id = lax.axis_index("s")

    def zfill(i, carry):
        zero_v[pl.ds(i * 16, 16)] = jnp.zeros((16,), F32)
        return carry
    lax.fori_loop(0, _SHARE // 16, zfill, 0)
    for j in range(_NROW):
        ones2[j, :] = jnp.ones((128,), F32).reshape(128,)

    for w in range(_NW):
        z1 = pltpu.async_copy(zero_v, adj_sh.at[pl.ds(s_id * _SHARE, _SHARE)], sem_z)
        z2 = pltpu.async_copy(zero_v, c_sh.at[pl.ds(s_id * _SHARE, _SHARE)], sem_z)

        g_local = s_id // (16 // _GPW)
        part = s_id % (16 // _GPW)
        g = c_id * 16 + w * _GPW + g_local
        estart = g * _EPG + part * _EPT
        st1 = pltpu.async_copy(src_hbm.at[pl.ds(estart, _EPT)], src_v, sem_st)
        st2 = pltpu.async_copy(dst_hbm.at[pl.ds(estart, _EPT)], dst_v, sem_st)
        st3 = pltpu.async_copy(ea_hbm.at[pl.ds(estart, _EPT)], ea_v, sem_st)
        st1.wait(); st2.wait(); st3.wait()

        base = g_local * (P * P)
        for kk in range(_EPT // 16):
            sv = src_v[pl.ds(kk * 16, 16)]
            dv = dst_v[pl.ds(kk * 16, 16)]
            il = base + (sv & (P - 1)) * P + (dv & (P - 1))
            j, col = kk // 8, (kk % 8) * 16
            idx2[j, pl.ds(col, 16)] = il
            val2[j, pl.ds(col, 16)] = ea_v[pl.ds(kk * 16, 16)]

        z1.wait(); z2.wait()
        plsc.subcore_barrier()

        descs = []
        for j in range(_NROW):
            descs.append(pltpu.async_copy(
                val2.at[j], adj_sh.at[idx2.at[j]], sem_sc, add=True))
            descs.append(pltpu.async_copy(
                ones2.at[j], c_sh.at[idx2.at[j]], sem_sc, add=True))
        for d in descs:
            d.wait()
        plsc.subcore_barrier()

        out_base = (c_id * 16 + w * _GPW) * (P * P) + s_id * _SHARE
        d1 = pltpu.async_copy(adj_sh.at[pl.ds(s_id * _SHARE, _SHARE)],
                              adj_hbm.at[pl.ds(out_base, _SHARE)], sem_d)
        d2 = pltpu.async_copy(c_sh.at[pl.ds(s_id * _SHARE, _SHARE)],
                              c_hbm.at[pl.ds(out_base, _SHARE)], sem_d)
        d1.wait(); d2.wait()
        plsc.subcore_barrier()


def _sc_build_adj(src, dst, ea):
    k = pl.kernel(
        _sc_scatter_body,
        out_type=[jax.ShapeDtypeStruct((B * P * P,), F32),
                  jax.ShapeDtypeStruct((B * P * P,), F32)],
        mesh=plsc.VectorSubcoreMesh(core_axis_name="c", subcore_axis_name="s"),
        scratch_types=[
            pltpu.VMEM((_EPT,), jnp.int32),
            pltpu.VMEM((_EPT,), jnp.int32),
            pltpu.VMEM((_EPT,), F32),
            pltpu.VMEM((_NROW, 128), jnp.int32),
            pltpu.VMEM((_NROW, 128), F32),
            pltpu.VMEM((_NROW, 128), F32),
            pltpu.VMEM((_SHARE,), F32),
            pltpu.VMEM_SHARED((_WORDS,), F32),
            pltpu.VMEM_SHARED((_WORDS,), F32),
            pltpu.SemaphoreType.DMA,
            pltpu.SemaphoreType.DMA,
            pltpu.SemaphoreType.DMA,
            pltpu.SemaphoreType.DMA,
        ],
    )
    adj_flat, c_flat = k(src, dst, ea)
    return adj_flat.reshape(B, P, P), c_flat.reshape(B, P, P)
